# baseline (speedup 1.0000x reference)
import functools

import jax
import jax.numpy as jnp
from jax import lax
from jax.experimental import pallas as pl
from jax.experimental.pallas import tpu as pltpu
from jax.experimental.pallas import tpu_sc as plsc

_B, _C, _H, _W = 4, 96, 224, 224
_OH, _OW = 2 * _H, 2 * _W
_K = 48
_NW = 32
_NTASK = _B * _OH
_TPW = _NTASK // _NW
_INV = float(1.0 / (_OH - 1))
_CHUNK = 64


def _sd(v):
    k, _ = plsc.sort_key_val(v, v, descending=True)
    return k


def _sa(v):
    k, _ = plsc.sort_key_val(v, v, descending=False)
    return k


def _top48(v):
    def m_desc(a, b):
        hi = jnp.maximum(a, b)
        lo = jnp.minimum(a, b)
        return [_sd(hi), _sd(lo)]

    def m_asc(a, b):
        hi = jnp.maximum(a, b)
        lo = jnp.minimum(a, b)
        return [_sa(lo), _sa(hi)]

    S1 = m_desc(_sd(v[0]), _sa(v[1]))
    S2 = m_asc(_sd(v[2]), _sa(v[3]))
    S3 = m_asc(_sd(v[4]), _sa(v[5]))
    h0 = jnp.maximum(S1[0], S2[0])
    h1 = jnp.maximum(S1[1], S2[1])
    l0 = jnp.minimum(S1[0], S2[0])
    l1 = jnp.minimum(S1[1], S2[1])
    t0 = _sd(jnp.maximum(h0, h1))
    t1 = _sd(jnp.minimum(h0, h1))
    t2 = _sd(jnp.maximum(l0, l1))
    t3 = _sd(jnp.minimum(l0, l1))
    h2 = jnp.maximum(t2, S3[0])
    h3 = jnp.maximum(t3, S3[1])
    p0 = jnp.maximum(t0, h2)
    p1 = jnp.maximum(t1, h3)
    p2 = jnp.minimum(t0, h2)
    p3 = jnp.minimum(t1, h3)
    q0 = jnp.maximum(p0, p1)
    q1 = jnp.minimum(p0, p1)
    q2 = jnp.maximum(p2, p3)
    return [_sd(q0), _sd(q1), _sd(q2)]


def _body(x_hbm, out_hbm, rows_v, interp_v, out_v, sem_in, sem_out):
    cid = lax.axis_index("c")
    sid = lax.axis_index("s")
    wid = sid * 2 + cid
    lane = lax.iota(jnp.int32, 16)

    def task_idx(j):
        t = wid * _TPW + j
        b = t // _OH
        r = t - b * _OH
        ynum = r * (_H - 1)
        y0 = ynum // (_OH - 1)
        wy = (ynum - y0 * (_OH - 1)).astype(jnp.float32) * _INV
        y0c = jnp.minimum(y0, _H - 2)
        wyc = wy + (y0 - y0c).astype(jnp.float32)
        return b, r, y0c, wyc

    b0, _, y0c0, _ = task_idx(0)
    pltpu.async_copy(x_hbm.at[b0, :, pl.ds(y0c0, 2), :], rows_v.at[0], sem_in)

    def task_body(j, carry):
        p = jnp.bitwise_and(j, 1)
        b, r, y0c, wy = task_idx(j)
        pltpu.make_async_copy(
            x_hbm.at[b, :, pl.ds(y0c, 2), :], rows_v.at[p], sem_in
        ).wait()

        @pl.when(j + 1 < _TPW)
        def _():
            bn, _, y0cn, _ = task_idx(j + 1)
            pltpu.async_copy(
                x_hbm.at[bn, :, pl.ds(y0cn, 2), :], rows_v.at[1 - p], sem_in
            )

        wyv = jnp.full((16,), wy, jnp.float32)

        @plsc.parallel_loop(0, _C, 1, unroll=2)
        def interp_body(c):
            for jj in range(_W // 16):
                a = rows_v[p, c, 0, pl.ds(jj * 16, 16)]
                bb = rows_v[p, c, 1, pl.ds(jj * 16, 16)]
                interp_v[c, pl.ds(jj * 16, 16)] = a + (bb - a) * wyv

        @pl.when(j > 0)
        def _():
            bp, rp, _, _ = task_idx(j - 1)
            pltpu.make_async_copy(
                out_v.at[:, pl.ds(0, _OW)], out_hbm.at[bp, :, rp, :], sem_out
            ).wait()

        @plsc.parallel_loop(0, _OW, 1, unroll=3)
        def pix_body(ow):
            xn = ow * (_W - 1)
            x0 = xn // (_OW - 1)
            wx = (xn - x0 * (_OW - 1)).astype(jnp.float32) * _INV
            x1 = jnp.minimum(x0 + 1, _W - 1)
            wxv = jnp.full((16,), wx, jnp.float32)
            x0v = jnp.full((16,), x0, jnp.int32)
            x1v = jnp.full((16,), x1, jnp.int32)
            vals = []
            for g in range(6):
                cvec = lane + (16 * g)
                a0 = plsc.load_gather(interp_v, [cvec, x0v])
                a1 = plsc.load_gather(interp_v, [cvec, x1v])
                vals.append(a0 + (a1 - a0) * wxv)
            o = _top48(vals)
            owv = jnp.full((16,), ow, jnp.int32)
            for k3 in range(3):
                plsc.store_scatter(out_v, [lane + 16 * k3, owv], o[k3])

        pltpu.async_copy(
            out_v.at[:, pl.ds(0, _OW)], out_hbm.at[b, :, r, :], sem_out
        )
        return carry

    lax.fori_loop(0, _TPW, task_body, 0)
    bl, rl, _, _ = task_idx(_TPW - 1)
    pltpu.make_async_copy(
        out_v.at[:, pl.ds(0, _OW)], out_hbm.at[bl, :, rl, :], sem_out
    ).wait()


@functools.partial(
    pl.kernel,
    out_type=jax.ShapeDtypeStruct((_B, _K, _OH, _OW), jnp.float32),
    mesh=plsc.VectorSubcoreMesh(core_axis_name="c", subcore_axis_name="s"),
    scratch_types=[
        pltpu.VMEM((2, _C, 2, _W), jnp.float32),
        pltpu.VMEM((_C, _W + 1), jnp.float32),
        pltpu.VMEM((_K, _OW + 1), jnp.float32),
        pltpu.SemaphoreType.DMA,
        pltpu.SemaphoreType.DMA,
    ],
    compiler_params=pltpu.CompilerParams(
        use_tc_tiling_on_sc=False, needs_layout_passes=False
    ),
)
def _run(x_hbm, out_hbm, rows_v, interp_v, out_v, sem_in, sem_out):
    _body(x_hbm, out_hbm, rows_v, interp_v, out_v, sem_in, sem_out)


def kernel(x):
    return _run(x)


# speedup vs baseline: 1.1966x; 1.1966x over previous
import functools

import jax
import jax.numpy as jnp
from jax import lax
from jax.experimental import pallas as pl
from jax.experimental.pallas import tpu as pltpu
from jax.experimental.pallas import tpu_sc as plsc

_B, _C, _H, _W = 4, 96, 224, 224
_OH, _OW = 2 * _H, 2 * _W
_K = 48
_NW = 32
_NTASK = _B * _OH
_TPW = _NTASK // _NW
_INV = float(1.0 / (_OH - 1))
_CHUNK = 64


def _sd(v):
    k, _ = plsc.sort_key_val(v, v, descending=True)
    return k


def _sa(v):
    k, _ = plsc.sort_key_val(v, v, descending=False)
    return k


def _top48(v):
    def m_desc(a, b):
        hi = jnp.maximum(a, b)
        lo = jnp.minimum(a, b)
        return [_sd(hi), _sd(lo)]

    def m_asc(a, b):
        hi = jnp.maximum(a, b)
        lo = jnp.minimum(a, b)
        return [_sa(lo), _sa(hi)]

    S1 = m_desc(_sd(v[0]), _sa(v[1]))
    S2 = m_asc(_sd(v[2]), _sa(v[3]))
    S3 = m_asc(_sd(v[4]), _sa(v[5]))
    h0 = jnp.maximum(S1[0], S2[0])
    h1 = jnp.maximum(S1[1], S2[1])
    l0 = jnp.minimum(S1[0], S2[0])
    l1 = jnp.minimum(S1[1], S2[1])
    t0 = _sd(jnp.maximum(h0, h1))
    t1 = _sd(jnp.minimum(h0, h1))
    t2 = _sd(jnp.maximum(l0, l1))
    t3 = _sd(jnp.minimum(l0, l1))
    h2 = jnp.maximum(t2, S3[0])
    h3 = jnp.maximum(t3, S3[1])
    p0 = jnp.maximum(t0, h2)
    p1 = jnp.maximum(t1, h3)
    p2 = jnp.minimum(t0, h2)
    p3 = jnp.minimum(t1, h3)
    q0 = jnp.maximum(p0, p1)
    q1 = jnp.minimum(p0, p1)
    q2 = jnp.maximum(p2, p3)
    return [_sd(q0), _sd(q1), _sd(q2)]


def _body(x_hbm, out_hbm, rows_v, interp_v, out_v, sem_in, sem_out):
    cid = lax.axis_index("c")
    sid = lax.axis_index("s")
    wid = sid * 2 + cid
    lane = lax.iota(jnp.int32, 16)

    def task_idx(j):
        t = wid * _TPW + j
        b = t // _OH
        r = t - b * _OH
        ynum = r * (_H - 1)
        y0 = ynum // (_OH - 1)
        wy = (ynum - y0 * (_OH - 1)).astype(jnp.float32) * _INV
        y0c = jnp.minimum(y0, _H - 2)
        wyc = wy + (y0 - y0c).astype(jnp.float32)
        return b, r, y0c, wyc

    b0, _, y0c0, _ = task_idx(0)
    pltpu.async_copy(x_hbm.at[b0, :, pl.ds(y0c0, 2), :], rows_v.at[0], sem_in)

    def task_body(j, carry):
        p = jnp.bitwise_and(j, 1)
        b, r, y0c, wy = task_idx(j)
        pltpu.make_async_copy(
            x_hbm.at[b, :, pl.ds(y0c, 2), :], rows_v.at[p], sem_in
        ).wait()

        @pl.when(j + 1 < _TPW)
        def _():
            bn, _, y0cn, _ = task_idx(j + 1)
            pltpu.async_copy(
                x_hbm.at[bn, :, pl.ds(y0cn, 2), :], rows_v.at[1 - p], sem_in
            )

        wyv = jnp.full((16,), wy, jnp.float32)

        @plsc.parallel_loop(0, _C, 1, unroll=2)
        def interp_body(c):
            for jj in range(_W // 16):
                a = rows_v[p, c, 0, pl.ds(jj * 16, 16)]
                bb = rows_v[p, c, 1, pl.ds(jj * 16, 16)]
                interp_v[c, pl.ds(jj * 16, 16)] = a + (bb - a) * wyv

        @pl.when(j > 0)
        def _():
            bp, rp, _, _ = task_idx(j - 1)
            pltpu.make_async_copy(
                out_v.at[:, pl.ds(0, _OW)], out_hbm.at[bp, :, rp, :], sem_out
            ).wait()

        @plsc.parallel_loop(0, _OW, 1, unroll=2)
        def pix_body(ow):
            xn = ow * (_W - 1)
            x0 = xn // (_OW - 1)
            wx = (xn - x0 * (_OW - 1)).astype(jnp.float32) * _INV
            x1 = jnp.minimum(x0 + 1, _W - 1)
            wxv = jnp.full((16,), wx, jnp.float32)
            x0v = jnp.full((16,), x0, jnp.int32)
            x1v = jnp.full((16,), x1, jnp.int32)
            vals = []
            for g in range(6):
                cvec = lane + (16 * g)
                a0 = plsc.load_gather(interp_v, [cvec, x0v])
                a1 = plsc.load_gather(interp_v, [cvec, x1v])
                vals.append(a0 + (a1 - a0) * wxv)
            o = _top48(vals)
            owv = jnp.full((16,), ow, jnp.int32)
            for k3 in range(3):
                plsc.store_scatter(out_v, [lane + 16 * k3, owv], o[k3])

        pltpu.async_copy(
            out_v.at[:, pl.ds(0, _OW)], out_hbm.at[b, :, r, :], sem_out
        )
        return carry

    lax.fori_loop(0, _TPW, task_body, 0)
    bl, rl, _, _ = task_idx(_TPW - 1)
    pltpu.make_async_copy(
        out_v.at[:, pl.ds(0, _OW)], out_hbm.at[bl, :, rl, :], sem_out
    ).wait()


@functools.partial(
    pl.kernel,
    out_type=jax.ShapeDtypeStruct((_B, _K, _OH, _OW), jnp.float32),
    mesh=plsc.VectorSubcoreMesh(core_axis_name="c", subcore_axis_name="s"),
    scratch_types=[
        pltpu.VMEM((2, _C, 2, _W), jnp.float32),
        pltpu.VMEM((_C, _W + 1), jnp.float32),
        pltpu.VMEM((_K, _OW + 1), jnp.float32),
        pltpu.SemaphoreType.DMA,
        pltpu.SemaphoreType.DMA,
    ],
    compiler_params=pltpu.CompilerParams(
        use_tc_tiling_on_sc=False, needs_layout_passes=False
    ),
)
def _run(x_hbm, out_hbm, rows_v, interp_v, out_v, sem_in, sem_out):
    _body(x_hbm, out_hbm, rows_v, interp_v, out_v, sem_in, sem_out)


def kernel(x):
    return _run(x)


# speedup vs baseline: 1.1976x; 1.0008x over previous
import functools

import jax
import jax.numpy as jnp
from jax import lax
from jax.experimental import pallas as pl
from jax.experimental.pallas import tpu as pltpu
from jax.experimental.pallas import tpu_sc as plsc

_B, _C, _H, _W = 4, 96, 224, 224
_OH, _OW = 2 * _H, 2 * _W
_K = 48
_NW = 32
_NTASK = _B * _OH
_TPW = _NTASK // _NW
_INV = float(1.0 / (_OH - 1))
_CHUNK = 64


def _sd(v):
    k, _ = plsc.sort_key_val(v, v, descending=True)
    return k


def _sa(v):
    k, _ = plsc.sort_key_val(v, v, descending=False)
    return k


def _top48(v):
    def m_desc(a, b):
        hi = jnp.maximum(a, b)
        lo = jnp.minimum(a, b)
        return [_sd(hi), _sd(lo)]

    def m_asc(a, b):
        hi = jnp.maximum(a, b)
        lo = jnp.minimum(a, b)
        return [_sa(lo), _sa(hi)]

    S1 = m_desc(_sd(v[0]), _sa(v[1]))
    S2 = m_asc(_sd(v[2]), _sa(v[3]))
    S3 = m_asc(_sd(v[4]), _sa(v[5]))
    h0 = jnp.maximum(S1[0], S2[0])
    h1 = jnp.maximum(S1[1], S2[1])
    l0 = jnp.minimum(S1[0], S2[0])
    l1 = jnp.minimum(S1[1], S2[1])
    t0 = _sd(jnp.maximum(h0, h1))
    t1 = _sd(jnp.minimum(h0, h1))
    t2 = _sd(jnp.maximum(l0, l1))
    t3 = _sd(jnp.minimum(l0, l1))
    h2 = jnp.maximum(t2, S3[0])
    h3 = jnp.maximum(t3, S3[1])
    p0 = jnp.maximum(t0, h2)
    p1 = jnp.maximum(t1, h3)
    p2 = jnp.minimum(t0, h2)
    p3 = jnp.minimum(t1, h3)
    q0 = jnp.maximum(p0, p1)
    q1 = jnp.minimum(p0, p1)
    q2 = jnp.maximum(p2, p3)
    return [_sd(q0), _sd(q1), _sd(q2)]


def _body(x_hbm, out_hbm, rows_v, interp_v, out_v, sem_in, sem_out):
    cid = lax.axis_index("c")
    sid = lax.axis_index("s")
    wid = sid * 2 + cid
    lane = lax.iota(jnp.int32, 16)

    def task_idx(j):
        t = wid * _TPW + j
        b = t // _OH
        r = t - b * _OH
        ynum = r * (_H - 1)
        y0 = ynum // (_OH - 1)
        wy = (ynum - y0 * (_OH - 1)).astype(jnp.float32) * _INV
        y0c = jnp.minimum(y0, _H - 2)
        wyc = wy + (y0 - y0c).astype(jnp.float32)
        return b, r, y0c, wyc

    b0, _, y0c0, _ = task_idx(0)
    pltpu.async_copy(x_hbm.at[b0, :, pl.ds(y0c0, 2), :], rows_v.at[0], sem_in)

    def task_body(j, carry):
        p = jnp.bitwise_and(j, 1)
        b, r, y0c, wy = task_idx(j)
        pltpu.make_async_copy(
            x_hbm.at[b, :, pl.ds(y0c, 2), :], rows_v.at[p], sem_in
        ).wait()

        @pl.when(j + 1 < _TPW)
        def _():
            bn, _, y0cn, _ = task_idx(j + 1)
            pltpu.async_copy(
                x_hbm.at[bn, :, pl.ds(y0cn, 2), :], rows_v.at[1 - p], sem_in
            )

        wyv = jnp.full((16,), wy, jnp.float32)

        @plsc.parallel_loop(0, _C, 1, unroll=2)
        def interp_body(c):
            for jj in range(_W // 16):
                a = rows_v[p, c, 0, pl.ds(jj * 16, 16)]
                bb = rows_v[p, c, 1, pl.ds(jj * 16, 16)]
                interp_v[c, pl.ds(jj * 16, 16)] = a + (bb - a) * wyv

        @pl.when(j > 0)
        def _():
            bp, rp, _, _ = task_idx(j - 1)
            pltpu.make_async_copy(
                out_v.at[:, pl.ds(0, _OW)], out_hbm.at[bp, :, rp, :], sem_out
            ).wait()

        @plsc.parallel_loop(0, _OW, 1, unroll=2)
        def pix_body(ow):
            xn = ow * (_W - 1)
            x0 = xn // (_OW - 1)
            wx = (xn - x0 * (_OW - 1)).astype(jnp.float32) * _INV
            x1 = jnp.minimum(x0 + 1, _W - 1)
            wxv = jnp.full((16,), wx, jnp.float32)
            x0v = jnp.full((16,), x0, jnp.int32)
            x1v = jnp.full((16,), x1, jnp.int32)
            vals = []
            for g in range(6):
                cvec = lane + (16 * g)
                a0 = plsc.load_gather(interp_v, [cvec, x0v])
                a1 = plsc.load_gather(interp_v, [cvec, x1v])
                vals.append(a0 + (a1 - a0) * wxv)
            o = _top48(vals)
            owv = jnp.full((16,), ow, jnp.int32)
            for k3 in range(3):
                plsc.store_scatter(out_v, [lane + 16 * k3, owv], o[k3])

        pltpu.async_copy(
            out_v.at[:, pl.ds(0, _OW)], out_hbm.at[b, :, r, :], sem_out
        )
        return carry

    lax.fori_loop(0, _TPW, task_body, 0)
    bl, rl, _, _ = task_idx(_TPW - 1)
    pltpu.make_async_copy(
        out_v.at[:, pl.ds(0, _OW)], out_hbm.at[bl, :, rl, :], sem_out
    ).wait()


@functools.partial(
    pl.kernel,
    out_type=jax.ShapeDtypeStruct((_B, _K, _OH, _OW), jnp.float32),
    mesh=plsc.VectorSubcoreMesh(core_axis_name="c", subcore_axis_name="s"),
    scratch_types=[
        pltpu.VMEM((2, _C, 2, _W), jnp.float32),
        pltpu.VMEM((_C, _W + 1), jnp.float32),
        pltpu.VMEM((_K, _OW + 1), jnp.float32),
        pltpu.SemaphoreType.DMA,
        pltpu.SemaphoreType.DMA,
    ],
    compiler_params=pltpu.CompilerParams(
        use_tc_tiling_on_sc=False, needs_layout_passes=False, skip_device_barrier=True
    ),
)
def _run(x_hbm, out_hbm, rows_v, interp_v, out_v, sem_in, sem_out):
    _body(x_hbm, out_hbm, rows_v, interp_v, out_v, sem_in, sem_out)


def kernel(x):
    return _run(x)
